# baseline (speedup 1.0000x reference)
import functools

import jax
import jax.numpy as jnp
from jax import lax
from jax.experimental import pallas as pl
from jax.experimental.pallas import tpu as pltpu
from jax.experimental.pallas import tpu_sc as plsc

_N = 2000000
_NC = 2
_NS = 16
_NW = _NC * _NS
_CHUNK = (_N // (_NW * 8)) * 8
_TAIL = _N - _NW * _CHUNK


def _tc_copy(arrs):
    n = len(arrs)

    def body(*refs):
        ins = refs[:n]
        outs = refs[n:2 * n]
        sem = refs[2 * n]
        cps = [pltpu.make_async_copy(i, o, sem) for i, o in zip(ins, outs)]
        for cp in cps:
            cp.start()
        for cp in cps:
            cp.wait()

    return pl.pallas_call(
        body,
        in_specs=[pl.BlockSpec(memory_space=pltpu.MemorySpace.HBM)] * n,
        out_specs=[pl.BlockSpec(memory_space=pltpu.MemorySpace.HBM)] * n,
        out_shape=[jax.ShapeDtypeStruct(a.shape, a.dtype) for a in arrs],
        scratch_shapes=[pltpu.SemaphoreType.DMA],
    )(*arrs)


def _sc_copy(arrs):
    n = len(arrs)
    mesh = plsc.VectorSubcoreMesh(core_axis_name="c", subcore_axis_name="s")

    scratch = [pltpu.VMEM((_CHUNK,), a.dtype) for a in arrs]
    scratch += [pltpu.SemaphoreType.DMA, pltpu.SemaphoreType.DMA]

    @functools.partial(
        pl.kernel,
        mesh=mesh,
        out_type=[jax.ShapeDtypeStruct(a.shape, a.dtype) for a in arrs],
        scratch_types=scratch,
    )
    def sc_kernel(*refs):
        ins = refs[:n]
        outs = refs[n:2 * n]
        bufs = refs[2 * n:3 * n]
        sin = refs[3 * n]
        sout = refs[3 * n + 1]
        w = lax.axis_index("s") * _NC + lax.axis_index("c")
        base = w * _CHUNK

        incps = [
            pltpu.make_async_copy(ins[a].at[pl.ds(base, _CHUNK)], bufs[a], sin)
            for a in range(n)
        ]
        for cp in incps:
            cp.start()
        for cp in incps:
            cp.wait()
        outcps = [
            pltpu.make_async_copy(bufs[a], outs[a].at[pl.ds(base, _CHUNK)], sout)
            for a in range(n)
        ]
        for cp in outcps:
            cp.start()
        for cp in outcps:
            cp.wait()

        @pl.when(w == _NW - 1)
        def _tail():
            tbase = _NW * _CHUNK
            for a in range(n):
                tin = pltpu.make_async_copy(
                    ins[a].at[pl.ds(tbase, _TAIL)],
                    bufs[a].at[pl.ds(0, _TAIL)], sin)
                tin.start()
                tin.wait()
                tout = pltpu.make_async_copy(
                    bufs[a].at[pl.ds(0, _TAIL)],
                    outs[a].at[pl.ds(tbase, _TAIL)], sout)
                tout.start()
                tout.wait()

    return sc_kernel(*arrs)


def kernel(xyz, colors, rots, scales, z_values, trackable_filter,
           new_xyz, new_colors, new_rots, new_scales, new_z_values,
           new_trackable_filter):
    t_out = _tc_copy((new_xyz.T, new_colors.T, new_rots.T, new_scales.T))
    s_out = _sc_copy((new_z_values, new_trackable_filter))
    return (t_out[0].T, t_out[1].T, t_out[2].T, t_out[3].T,
            s_out[0], s_out[1])


# speedup vs baseline: 1.0003x; 1.0003x over previous
import functools

import jax
import jax.numpy as jnp
from jax import lax
from jax.experimental import pallas as pl
from jax.experimental.pallas import tpu as pltpu
from jax.experimental.pallas import tpu_sc as plsc

_N = 2000000
_NC = 2
_NS = 16
_NW = _NC * _NS
_CHUNK = (_N // (_NW * 8)) * 8
_TAIL = _N - _NW * _CHUNK


_N_SLICES = 125
_SLICE_COLS = _N // _N_SLICES


def _tc_copy(arrs):
    n = len(arrs)

    def body(*refs):
        ins = refs[:n]
        outs = refs[n:2 * n]
        sem = refs[2 * n]
        cps = []
        for i, o in zip(ins, outs):
            for k in range(_N_SLICES):
                cps.append(pltpu.make_async_copy(
                    i.at[:, pl.ds(k * _SLICE_COLS, _SLICE_COLS)],
                    o.at[:, pl.ds(k * _SLICE_COLS, _SLICE_COLS)],
                    sem))
        for cp in cps:
            cp.start()
        for cp in cps:
            cp.wait()

    return pl.pallas_call(
        body,
        in_specs=[pl.BlockSpec(memory_space=pltpu.MemorySpace.HBM)] * n,
        out_specs=[pl.BlockSpec(memory_space=pltpu.MemorySpace.HBM)] * n,
        out_shape=[jax.ShapeDtypeStruct(a.shape, a.dtype) for a in arrs],
        scratch_shapes=[pltpu.SemaphoreType.DMA],
    )(*arrs)


def _sc_copy(arrs):
    n = len(arrs)
    mesh = plsc.VectorSubcoreMesh(core_axis_name="c", subcore_axis_name="s")

    scratch = [pltpu.VMEM((_CHUNK,), a.dtype) for a in arrs]
    scratch += [pltpu.SemaphoreType.DMA, pltpu.SemaphoreType.DMA]

    @functools.partial(
        pl.kernel,
        mesh=mesh,
        out_type=[jax.ShapeDtypeStruct(a.shape, a.dtype) for a in arrs],
        scratch_types=scratch,
    )
    def sc_kernel(*refs):
        ins = refs[:n]
        outs = refs[n:2 * n]
        bufs = refs[2 * n:3 * n]
        sin = refs[3 * n]
        sout = refs[3 * n + 1]
        w = lax.axis_index("s") * _NC + lax.axis_index("c")
        base = w * _CHUNK

        incps = [
            pltpu.make_async_copy(ins[a].at[pl.ds(base, _CHUNK)], bufs[a], sin)
            for a in range(n)
        ]
        for cp in incps:
            cp.start()
        for cp in incps:
            cp.wait()
        outcps = [
            pltpu.make_async_copy(bufs[a], outs[a].at[pl.ds(base, _CHUNK)], sout)
            for a in range(n)
        ]
        for cp in outcps:
            cp.start()
        for cp in outcps:
            cp.wait()

        @pl.when(w == _NW - 1)
        def _tail():
            tbase = _NW * _CHUNK
            for a in range(n):
                tin = pltpu.make_async_copy(
                    ins[a].at[pl.ds(tbase, _TAIL)],
                    bufs[a].at[pl.ds(0, _TAIL)], sin)
                tin.start()
                tin.wait()
                tout = pltpu.make_async_copy(
                    bufs[a].at[pl.ds(0, _TAIL)],
                    outs[a].at[pl.ds(tbase, _TAIL)], sout)
                tout.start()
                tout.wait()

    return sc_kernel(*arrs)


def kernel(xyz, colors, rots, scales, z_values, trackable_filter,
           new_xyz, new_colors, new_rots, new_scales, new_z_values,
           new_trackable_filter):
    t_out = _tc_copy((new_xyz.T, new_colors.T, new_rots.T, new_scales.T))
    s_out = _sc_copy((new_z_values, new_trackable_filter))
    return (t_out[0].T, t_out[1].T, t_out[2].T, t_out[3].T,
            s_out[0], s_out[1])


# speedup vs baseline: 26.4241x; 26.4167x over previous
import functools

import jax
import jax.numpy as jnp
from jax import lax
from jax.experimental import pallas as pl
from jax.experimental.pallas import tpu as pltpu
from jax.experimental.pallas import tpu_sc as plsc

_N = 2000000
_NC = 2
_NS = 16
_NW = _NC * _NS
_CHUNK = (_N // (_NW * 8)) * 8
_TAIL = _N - _NW * _CHUNK


_GRID = 125
_BLOCK_COLS = _N // _GRID


def _tc_copy(arrs):
    n = len(arrs)

    def body(*refs):
        ins = refs[:n]
        outs = refs[n:2 * n]
        for i, o in zip(ins, outs):
            o[...] = i[...]

    specs = [
        pl.BlockSpec((a.shape[0], _BLOCK_COLS), lambda i: (0, i))
        for a in arrs
    ]
    return pl.pallas_call(
        body,
        grid=(_GRID,),
        in_specs=specs,
        out_specs=specs,
        out_shape=[jax.ShapeDtypeStruct(a.shape, a.dtype) for a in arrs],
    )(*arrs)


def _sc_copy(arrs):
    n = len(arrs)
    mesh = plsc.VectorSubcoreMesh(core_axis_name="c", subcore_axis_name="s")

    scratch = [pltpu.VMEM((_CHUNK,), a.dtype) for a in arrs]
    scratch += [pltpu.SemaphoreType.DMA, pltpu.SemaphoreType.DMA]

    @functools.partial(
        pl.kernel,
        mesh=mesh,
        out_type=[jax.ShapeDtypeStruct(a.shape, a.dtype) for a in arrs],
        scratch_types=scratch,
    )
    def sc_kernel(*refs):
        ins = refs[:n]
        outs = refs[n:2 * n]
        bufs = refs[2 * n:3 * n]
        sin = refs[3 * n]
        sout = refs[3 * n + 1]
        w = lax.axis_index("s") * _NC + lax.axis_index("c")
        base = w * _CHUNK

        incps = [
            pltpu.make_async_copy(ins[a].at[pl.ds(base, _CHUNK)], bufs[a], sin)
            for a in range(n)
        ]
        for cp in incps:
            cp.start()
        for cp in incps:
            cp.wait()
        outcps = [
            pltpu.make_async_copy(bufs[a], outs[a].at[pl.ds(base, _CHUNK)], sout)
            for a in range(n)
        ]
        for cp in outcps:
            cp.start()
        for cp in outcps:
            cp.wait()

        @pl.when(w == _NW - 1)
        def _tail():
            tbase = _NW * _CHUNK
            for a in range(n):
                tin = pltpu.make_async_copy(
                    ins[a].at[pl.ds(tbase, _TAIL)],
                    bufs[a].at[pl.ds(0, _TAIL)], sin)
                tin.start()
                tin.wait()
                tout = pltpu.make_async_copy(
                    bufs[a].at[pl.ds(0, _TAIL)],
                    outs[a].at[pl.ds(tbase, _TAIL)], sout)
                tout.start()
                tout.wait()

    return sc_kernel(*arrs)


def kernel(xyz, colors, rots, scales, z_values, trackable_filter,
           new_xyz, new_colors, new_rots, new_scales, new_z_values,
           new_trackable_filter):
    t_out = _tc_copy((new_xyz.T, new_colors.T, new_rots.T, new_scales.T))
    s_out = _sc_copy((new_z_values, new_trackable_filter))
    return (t_out[0].T, t_out[1].T, t_out[2].T, t_out[3].T,
            s_out[0], s_out[1])


# speedup vs baseline: 36.3136x; 1.3743x over previous
import functools

import jax
import jax.numpy as jnp
from jax import lax
from jax.experimental import pallas as pl
from jax.experimental.pallas import tpu as pltpu
from jax.experimental.pallas import tpu_sc as plsc

_N = 2000000
_NC = 2
_NS = 16
_NW = _NC * _NS
_CHUNK = (_N // (_NW * 8)) * 8
_TAIL = _N - _NW * _CHUNK


_GRID = 25
_BLOCK_COLS = _N // _GRID


def _tc_copy(arrs):
    n = len(arrs)

    def body(*refs):
        ins = refs[:n]
        outs = refs[n:2 * n]
        for i, o in zip(ins, outs):
            o[...] = i[...]

    specs = [
        pl.BlockSpec((a.shape[0], _BLOCK_COLS), lambda i: (0, i))
        for a in arrs
    ]
    return pl.pallas_call(
        body,
        grid=(_GRID,),
        in_specs=specs,
        out_specs=specs,
        out_shape=[jax.ShapeDtypeStruct(a.shape, a.dtype) for a in arrs],
    )(*arrs)


def _sc_copy(arrs):
    n = len(arrs)
    mesh = plsc.VectorSubcoreMesh(core_axis_name="c", subcore_axis_name="s")

    scratch = [pltpu.VMEM((_CHUNK,), a.dtype) for a in arrs]
    scratch += [pltpu.SemaphoreType.DMA, pltpu.SemaphoreType.DMA]

    @functools.partial(
        pl.kernel,
        mesh=mesh,
        out_type=[jax.ShapeDtypeStruct(a.shape, a.dtype) for a in arrs],
        scratch_types=scratch,
    )
    def sc_kernel(*refs):
        ins = refs[:n]
        outs = refs[n:2 * n]
        bufs = refs[2 * n:3 * n]
        sin = refs[3 * n]
        sout = refs[3 * n + 1]
        w = lax.axis_index("s") * _NC + lax.axis_index("c")
        base = w * _CHUNK

        incps = [
            pltpu.make_async_copy(ins[a].at[pl.ds(base, _CHUNK)], bufs[a], sin)
            for a in range(n)
        ]
        for cp in incps:
            cp.start()
        for cp in incps:
            cp.wait()
        outcps = [
            pltpu.make_async_copy(bufs[a], outs[a].at[pl.ds(base, _CHUNK)], sout)
            for a in range(n)
        ]
        for cp in outcps:
            cp.start()
        for cp in outcps:
            cp.wait()

        @pl.when(w == _NW - 1)
        def _tail():
            tbase = _NW * _CHUNK
            for a in range(n):
                tin = pltpu.make_async_copy(
                    ins[a].at[pl.ds(tbase, _TAIL)],
                    bufs[a].at[pl.ds(0, _TAIL)], sin)
                tin.start()
                tin.wait()
                tout = pltpu.make_async_copy(
                    bufs[a].at[pl.ds(0, _TAIL)],
                    outs[a].at[pl.ds(tbase, _TAIL)], sout)
                tout.start()
                tout.wait()

    return sc_kernel(*arrs)


def kernel(xyz, colors, rots, scales, z_values, trackable_filter,
           new_xyz, new_colors, new_rots, new_scales, new_z_values,
           new_trackable_filter):
    t_out = _tc_copy((new_xyz.T, new_colors.T, new_rots.T, new_scales.T))
    s_out = _sc_copy((new_z_values, new_trackable_filter))
    return (t_out[0].T, t_out[1].T, t_out[2].T, t_out[3].T,
            s_out[0], s_out[1])
